# Initial kernel scaffold; baseline (speedup 1.0000x reference)
import jax
import jax.numpy as jnp
from jax.experimental import pallas as pl


def kernel(x, pos_table, gamma, beta):
    raise NotImplementedError("write your pallas kernel here")


# speedup vs baseline: 1.8676x; 1.8676x over previous
import functools

import jax
import jax.numpy as jnp
from jax.experimental import pallas as pl

EPS = 1e-5


def _ln_body(x_ref, pe_ref, g_ref, b_ref, o_ref):
    x = x_ref[...]
    pe = pe_ref[...]
    h = x + pe[:, None, :]
    mean = jnp.mean(h, axis=-1, keepdims=True)
    d = h - mean
    var = jnp.mean(d * d, axis=-1, keepdims=True)
    o_ref[...] = d * jax.lax.rsqrt(var + EPS) * g_ref[...] + b_ref[...]


@jax.jit
def kernel(x, pos_table, gamma, beta):
    S, B, D = x.shape
    S_BLK = 32
    grid = (S // S_BLK,)
    g3 = gamma.reshape(1, 1, D)
    b3 = beta.reshape(1, 1, D)
    return pl.pallas_call(
        _ln_body,
        grid=grid,
        in_specs=[
            pl.BlockSpec((S_BLK, B, D), lambda i: (i, 0, 0)),
            pl.BlockSpec((S_BLK, D), lambda i: (i, 0)),
            pl.BlockSpec((1, 1, D), lambda i: (0, 0, 0)),
            pl.BlockSpec((1, 1, D), lambda i: (0, 0, 0)),
        ],
        out_specs=pl.BlockSpec((S_BLK, B, D), lambda i: (i, 0, 0)),
        out_shape=jax.ShapeDtypeStruct((S, B, D), x.dtype),
    )(x, pos_table[:S], g3, b3)
